# baseline (speedup 1.0000x reference)
import functools

import jax
import jax.numpy as jnp
from jax import lax
from jax.experimental import pallas as pl
from jax.experimental.pallas import tpu as pltpu
from jax.experimental.pallas import tpu_sc as plsc

D = 128
_INTERPRET = False


def _block(n, candidates):
    for c in candidates:
        if n % c == 0:
            return c
    return n


def _edge_block(E):
    return _block(E, (2000, 1600, 800, 400, 200, 80, 40, 16, 8))


def _node_block(N):
    return _block(N, (400, 200, 100, 50, 40, 16, 8))


def _ln(x, w, b):
    mu = jnp.mean(x, axis=-1, keepdims=True)
    var = jnp.mean((x - mu) ** 2, axis=-1, keepdims=True)
    return (x - mu) * jax.lax.rsqrt(var + 1e-5) * w + b


def _vspec(blk, imap):
    return pl.BlockSpec(blk, imap)


def _wspec(shape):
    return pl.BlockSpec(shape, lambda *a: (0,) * len(shape))


def _edge_msg_l0_body(tidx, re, eq, tq, Wx, bg, m):
    eb = re.shape[0]
    gid = jax.lax.broadcasted_iota(jnp.int32, (eb, tidx.shape[1]), 0)
    gid = gid + pl.program_id(0) * eb
    onehot = (gid == tidx[...]).astype(jnp.float32)
    ef0 = jnp.dot(onehot, tq[...], preferred_element_type=jnp.float32)
    x = ef0 * re[...] + eq[...]
    gx = jnp.dot(x, Wx[...], preferred_element_type=jnp.float32) + bg[...]
    z = jax.nn.sigmoid(gx[:, :D])
    n = jnp.tanh(gx[:, 2 * D:])
    m[...] = z * n


def _edge_msg_l1_body(re, eq, ef, h, Wx, Wh, bg, m):
    x = ef[...] * re[...] + eq[...]
    gx = jnp.dot(x, Wx[...], preferred_element_type=jnp.float32) + bg[...]
    gh = jnp.dot(h[...], Wh[...], preferred_element_type=jnp.float32)
    z = jax.nn.sigmoid(gx[:, :D] + gh[:, :D])
    r = jax.nn.sigmoid(gx[:, D:2 * D] + gh[:, D:2 * D])
    n = jnp.tanh(gx[:, 2 * D:] + r * gh[:, 2 * D:])
    m[...] = (1.0 - z) * h[...] + z * n


def _edge_msg_l0(re, eq, tq, tidx, Wx, bg):
    E = re.shape[0]
    eb = _edge_block(E)
    grid = E // eb
    t = tidx.shape[1]
    return pl.pallas_call(
        _edge_msg_l0_body,
        grid=(grid,),
        in_specs=[
            _wspec((1, t)),
            _vspec((eb, D), lambda i: (i, 0)),
            _vspec((eb, D), lambda i: (i, 0)),
            _wspec((t, D)),
            _wspec((D, 3 * D)),
            _wspec((1, 3 * D)),
        ],
        out_specs=_vspec((eb, D), lambda i: (i, 0)),
        out_shape=jax.ShapeDtypeStruct((E, D), jnp.float32),
        interpret=_INTERPRET,
    )(tidx, re, eq, tq, Wx, bg)


def _edge_msg_l1(re, eq, ef, h, Wx, Wh, bg):
    E = re.shape[0]
    eb = _edge_block(E)
    grid = E // eb
    return pl.pallas_call(
        _edge_msg_l1_body,
        grid=(grid,),
        in_specs=[
            _vspec((eb, D), lambda i: (i, 0)),
            _vspec((eb, D), lambda i: (i, 0)),
            _vspec((eb, D), lambda i: (i, 0)),
            _vspec((eb, D), lambda i: (i, 0)),
            _wspec((D, 3 * D)),
            _wspec((D, 3 * D)),
            _wspec((1, 3 * D)),
        ],
        out_specs=_vspec((eb, D), lambda i: (i, 0)),
        out_shape=jax.ShapeDtypeStruct((E, D), jnp.float32),
        interpret=_INTERPRET,
    )(re, eq, ef, h, Wx, Wh, bg)


def _pna_body(ssum, smax, smin, degr, avgr, prev, Wp, bp, lnw, lnb, raw, post):
    deg = degr[...]
    degc = jnp.maximum(deg, 1.0)
    pos = deg > 0
    mean = ssum[...] / degc
    mx = jnp.where(pos, smax[...], 0.0)
    mn = jnp.where(pos, smin[...], 0.0)
    agg = jnp.concatenate([mean, mx, mn], axis=1)
    ld = jnp.log(deg + 1.0)
    avg = avgr[...]
    amp = ld / avg
    att = avg / jnp.where(ld > 0, ld, 1.0)
    h_agg = jnp.concatenate([agg, agg * amp, agg * att], axis=1)
    r = jnp.dot(h_agg, Wp[...], preferred_element_type=jnp.float32) + bp[...]
    raw[...] = r
    post[...] = prev[...] + _ln(r, lnw[...], lnb[...])


def _pna(ssum, smax, smin, deg, avg_d, prev, Wp, bp, lnw, lnb):
    N = ssum.shape[0]
    nb = _node_block(N)
    grid = N // nb
    return pl.pallas_call(
        _pna_body,
        grid=(grid,),
        in_specs=[
            _vspec((nb, D), lambda i: (i, 0)),
            _vspec((nb, D), lambda i: (i, 0)),
            _vspec((nb, D), lambda i: (i, 0)),
            _vspec((nb, 1), lambda i: (i, 0)),
            _wspec((1, 1)),
            _vspec((nb, D), lambda i: (i, 0)),
            _wspec((9 * D, D)),
            _wspec((1, D)),
            _wspec((1, D)),
            _wspec((1, D)),
        ],
        out_specs=[
            _vspec((nb, D), lambda i: (i, 0)),
            _vspec((nb, D), lambda i: (i, 0)),
        ],
        out_shape=[
            jax.ShapeDtypeStruct((N, D), jnp.float32),
            jax.ShapeDtypeStruct((N, D), jnp.float32),
        ],
        interpret=_INTERPRET,
    )(ssum, smax, smin, deg, avg_d, prev, Wp, bp, lnw, lnb)


def _edge_upd_common(ef, eq, ns, nd, Wl1, Wl2, bl, Wq1, Wq2, Wq3, bq, lnw, lnb,
                     efo, eqo):
    g4 = (jnp.dot(ns, Wl1, preferred_element_type=jnp.float32)
          + jnp.dot(nd, Wl2, preferred_element_type=jnp.float32) + bl)
    gi = g4[:, :D]
    gf = g4[:, D:2 * D]
    go = g4[:, 2 * D:3 * D]
    gc = g4[:, 3 * D:]
    c = jax.nn.sigmoid(gf) * ef + jax.nn.sigmoid(gi) * jnp.tanh(gc)
    efn = jax.nn.sigmoid(go) * jnp.tanh(c)
    eqn = jnp.tanh(jnp.dot(eq, Wq1, preferred_element_type=jnp.float32)
                   + jnp.dot(ns, Wq2, preferred_element_type=jnp.float32)
                   + jnp.dot(nd, Wq3, preferred_element_type=jnp.float32) + bq)
    efo[...] = ef + _ln(efn, lnw, lnb)
    eqo[...] = eq + _ln(eqn, lnw, lnb)


def _edge_upd_l0_body(tidx, eq, ns, nd, tq, Wl1, Wl2, bl, Wq1, Wq2, Wq3, bq,
                      lnw, lnb, efo, eqo):
    eb = eq.shape[0]
    gid = jax.lax.broadcasted_iota(jnp.int32, (eb, tidx.shape[1]), 0)
    gid = gid + pl.program_id(0) * eb
    onehot = (gid == tidx[...]).astype(jnp.float32)
    ef = jnp.dot(onehot, tq[...], preferred_element_type=jnp.float32)
    _edge_upd_common(ef, eq[...], ns[...], nd[...], Wl1[...], Wl2[...],
                     bl[...], Wq1[...], Wq2[...], Wq3[...], bq[...],
                     lnw[...], lnb[...], efo, eqo)


def _edge_upd_l1_body(ef, eq, ns, nd, Wl1, Wl2, bl, Wq1, Wq2, Wq3, bq,
                      lnw, lnb, efo, eqo):
    _edge_upd_common(ef[...], eq[...], ns[...], nd[...], Wl1[...], Wl2[...],
                     bl[...], Wq1[...], Wq2[...], Wq3[...], bq[...],
                     lnw[...], lnb[...], efo, eqo)


def _edge_upd(layer0, ef_or_tidx, eq, ns, nd, tq, Wl1, Wl2, bl, Wq1, Wq2,
              Wq3, bq, lnw, lnb):
    E = eq.shape[0]
    eb = _edge_block(E)
    grid = E // eb
    espec = _vspec((eb, D), lambda i: (i, 0))
    wspecs = [
        _wspec((D, 4 * D)), _wspec((D, 4 * D)), _wspec((1, 4 * D)),
        _wspec((D, D)), _wspec((D, D)), _wspec((D, D)), _wspec((1, D)),
        _wspec((1, D)), _wspec((1, D)),
    ]
    outs = dict(
        out_specs=[espec, espec],
        out_shape=[jax.ShapeDtypeStruct((E, D), jnp.float32),
                   jax.ShapeDtypeStruct((E, D), jnp.float32)],
        interpret=_INTERPRET,
    )
    if layer0:
        t = ef_or_tidx.shape[1]
        return pl.pallas_call(
            _edge_upd_l0_body,
            grid=(grid,),
            in_specs=[_wspec((1, t)), espec, espec, espec, _wspec((t, D))]
            + wspecs,
            **outs,
        )(ef_or_tidx, eq, ns, nd, tq, Wl1, Wl2, bl, Wq1, Wq2, Wq3, bq,
          lnw, lnb)
    return pl.pallas_call(
        _edge_upd_l1_body,
        grid=(grid,),
        in_specs=[espec, espec, espec, espec] + wspecs,
        **outs,
    )(ef_or_tidx, eq, ns, nd, Wl1, Wl2, bl, Wq1, Wq2, Wq3, bq, lnw, lnb)


def _head_body(tef, teq, tnf, Wej, bej, Wnj, bnj, Wqj, bqj, Wfc, u, v, w):
    ef32 = jnp.dot(tef[...], Wej[...], preferred_element_type=jnp.float32) + bej[...]
    eq32 = jnp.dot(teq[...], Wqj[...], preferred_element_type=jnp.float32) + bqj[...]
    nf32 = jnp.dot(tnf[...], Wnj[...], preferred_element_type=jnp.float32) + bnj[...]
    A = Wfc[:D, :]
    Bp = Wfc[D:2 * D, :]
    C = Wfc[2 * D:3 * D, :]
    Dp = Wfc[3 * D:, :]
    u[...] = (jnp.dot(ef32, A, preferred_element_type=jnp.float32)
              + jnp.dot(eq32, Bp, preferred_element_type=jnp.float32))
    v[...] = jnp.dot(nf32, C, preferred_element_type=jnp.float32)
    w[...] = jnp.dot(nf32, Dp, preferred_element_type=jnp.float32)


def _head(tef, teq, tnf, Wej, bej, Wnj, bnj, Wqj, bqj, Wfc):
    T = tef.shape[0]
    K = tef.shape[1]
    return pl.pallas_call(
        _head_body,
        in_specs=[
            _wspec((T, K)), _wspec((T, K)), _wspec((T, K)),
            _wspec((K, D)), _wspec((1, D)),
            _wspec((K, D)), _wspec((1, D)),
            _wspec((K, D)), _wspec((1, D)),
            _wspec((4 * D, 1)),
        ],
        out_specs=[_wspec((T, 1)), _wspec((T, 1)), _wspec((T, 1))],
        out_shape=[jax.ShapeDtypeStruct((T, 1), jnp.float32)] * 3,
        interpret=_INTERPRET,
    )(tef, teq, tnf, Wej, bej, Wnj, bnj, Wqj, bqj, Wfc)


def _gather_rows(table, idx):
    return jnp.take(table, idx, axis=0)


_GCH = 128


def _sc_mesh():
    return plsc.VectorSubcoreMesh(core_axis_name="c", subcore_axis_name="s")


def _sc_gather(table, idx):
    E2 = idx.shape[0]
    Dt = table.shape[1]
    assert E2 % _GCH == 0
    nch = E2 // _GCH
    info = plsc.get_sparse_core_info()
    NC, NS = info.num_cores, info.num_subcores
    NW = NC * NS
    bound = -(-nch // NW)

    @functools.partial(
        pl.kernel,
        mesh=_sc_mesh(),
        out_type=jax.ShapeDtypeStruct((E2, Dt), jnp.float32),
        scratch_types=[
            pltpu.VMEM((_GCH,), jnp.int32),
            pltpu.VMEM((_GCH, Dt), jnp.float32),
            pltpu.SemaphoreType.DMA,
        ],
    )
    def k(table_hbm, idx_hbm, out_hbm, idxb, rows, sem):
        wid = lax.axis_index("s") * NC + lax.axis_index("c")

        def body(j, carry):
            c = wid + j * NW

            @pl.when(c < nch)
            def _():
                base = c * _GCH
                pltpu.sync_copy(idx_hbm.at[pl.ds(base, _GCH)], idxb)
                pltpu.async_copy(table_hbm.at[idxb], rows, sem).wait()
                pltpu.sync_copy(rows, out_hbm.at[pl.ds(base, _GCH)])

            return carry

        lax.fori_loop(0, bound, body, 0)

    return k(table, idx)


def _segment_reduce(m, dst, N):
    ssum = jax.ops.segment_sum(m, dst, num_segments=N)
    smax = jax.ops.segment_max(m, dst, num_segments=N)
    smin = jax.ops.segment_min(m, dst, num_segments=N)
    return ssum, smax, smin


_SEG_PER = 320
_SEG_CHD = 400
_SEG_RCH = 48


def _sc_segment(m, dstv):
    E2 = m.shape[0]
    Dt = m.shape[1]
    info = plsc.get_sparse_core_info()
    NC = info.num_cores
    NW = NC * info.num_subcores
    PER = _SEG_PER
    Npad = NW * PER
    nch = E2 // _SEG_CHD
    NV = _SEG_CHD // 16
    NSUB = -(-_SEG_CHD // _SEG_RCH)
    FMAX = 3.4028235e38

    @functools.partial(
        pl.kernel,
        mesh=_sc_mesh(),
        compiler_params=pltpu.CompilerParams(needs_layout_passes=False),
        out_type=(jax.ShapeDtypeStruct((Npad, Dt), jnp.float32),
                  jax.ShapeDtypeStruct((Npad, Dt), jnp.float32),
                  jax.ShapeDtypeStruct((Npad, Dt), jnp.float32)),
        scratch_types=[
            pltpu.VMEM((_SEG_CHD,), jnp.int32),
            pltpu.VMEM((_SEG_CHD + _SEG_RCH,), jnp.int32),
            pltpu.VMEM((_SEG_CHD + 16,), jnp.int32),
            pltpu.VMEM((_SEG_RCH, Dt), jnp.float32),
            pltpu.VMEM((PER, Dt), jnp.float32),
            pltpu.VMEM((PER, Dt), jnp.float32),
            pltpu.VMEM((PER, Dt), jnp.float32),
            pltpu.SemaphoreType.DMA,
        ],
    )
    def k(m_hbm, dst_hbm, sum_hbm, max_hbm, min_hbm,
          dstb, idxb, slotb, rows, ssum, smax, smin, sem):
        wid = lax.axis_index("s") * NC + lax.axis_index("c")
        lo = wid * PER
        zero16 = jnp.zeros((16,), jnp.float32)
        ninf = jnp.full((16,), -FMAX, jnp.float32)
        pinf = jnp.full((16,), FMAX, jnp.float32)
        izero = jnp.zeros((16,), jnp.int32)
        iota = lax.iota(jnp.int32, 16)

        def init_row(i, c):
            for kq in range(Dt // 16):
                ssum[i, pl.ds(kq * 16, 16)] = zero16
                smax[i, pl.ds(kq * 16, 16)] = ninf
                smin[i, pl.ds(kq * 16, 16)] = pinf
            return c

        lax.fori_loop(0, PER, init_row, 0)

        def init_idx(i, c):
            idxb[pl.ds(i * 16, 16)] = izero
            return c

        lax.fori_loop(0, (_SEG_CHD + _SEG_RCH) // 16, init_idx, 0)

        def chunk(c, carry):
            base = c * _SEG_CHD
            pltpu.sync_copy(dst_hbm.at[pl.ds(base, _SEG_CHD)], dstb)

            def scan(v, cnt):
                d = dstb[pl.ds(v * 16, 16)]
                slot = d - lo
                msk = (slot >= 0) & (slot < PER)
                eidx = base + v * 16 + iota
                plsc.store_compressed(idxb.at[pl.ds(cnt, 16)], eidx, mask=msk)
                plsc.store_compressed(slotb.at[pl.ds(cnt, 16)], slot, mask=msk)
                return cnt + jnp.sum(msk.astype(jnp.int32))

            total = lax.fori_loop(0, NV, scan, 0)

            def sub(g, c2):
                gb = g * _SEG_RCH

                @pl.when(gb < total)
                def _():
                    pltpu.async_copy(
                        m_hbm.at[idxb.at[pl.ds(gb, _SEG_RCH)]], rows, sem
                    ).wait()

                    def edge(j, c3):
                        @pl.when(gb + j < total)
                        def _():
                            slot = slotb[pl.ds(gb + j, 16)][0]
                            for kq in range(Dt // 16):
                                ds = pl.ds(kq * 16, 16)
                                rv = rows[j, ds]
                                ssum[slot, ds] = ssum[slot, ds] + rv
                                smax[slot, ds] = jnp.maximum(smax[slot, ds], rv)
                                smin[slot, ds] = jnp.minimum(smin[slot, ds], rv)

                        return c3

                    lax.fori_loop(0, _SEG_RCH, edge, 0)

                return c2

            lax.fori_loop(0, NSUB, sub, 0)
            return carry

        lax.fori_loop(0, nch, chunk, 0)

        pltpu.sync_copy(ssum, sum_hbm.at[pl.ds(lo, PER)])
        pltpu.sync_copy(smax, max_hbm.at[pl.ds(lo, PER)])
        pltpu.sync_copy(smin, min_hbm.at[pl.ds(lo, PER)])

    return k(m, dstv)


def _sc_counts(dstv, srcv):
    E2 = dstv.shape[0]
    info = plsc.get_sparse_core_info()
    NC = info.num_cores
    NW = NC * info.num_subcores
    PER = _SEG_PER
    Npad = NW * PER
    nch = E2 // _SEG_CHD
    NV = _SEG_CHD // 16

    @functools.partial(
        pl.kernel,
        mesh=_sc_mesh(),
        compiler_params=pltpu.CompilerParams(needs_layout_passes=False),
        out_type=(jax.ShapeDtypeStruct((Npad,), jnp.float32),
                  jax.ShapeDtypeStruct((Npad,), jnp.float32)),
        scratch_types=[
            pltpu.VMEM((_SEG_CHD,), jnp.int32),
            pltpu.VMEM((PER,), jnp.float32),
            pltpu.VMEM((PER,), jnp.float32),
            pltpu.SemaphoreType.DMA,
        ],
    )
    def k(dst_hbm, src_hbm, deg_hbm, odeg_hbm, idxc, dacc, oacc, sem):
        wid = lax.axis_index("s") * NC + lax.axis_index("c")
        lo = wid * PER
        zero16 = jnp.zeros((16,), jnp.float32)
        ones = jnp.ones((16,), jnp.float32)

        def init(i, c):
            dacc[pl.ds(i * 16, 16)] = zero16
            oacc[pl.ds(i * 16, 16)] = zero16
            return c

        lax.fori_loop(0, PER // 16, init, 0)

        def pass_(idx_hbm, acc):
            def chunk(c, carry):
                pltpu.sync_copy(idx_hbm.at[pl.ds(c * _SEG_CHD, _SEG_CHD)], idxc)

                def scan(v, c2):
                    d = idxc[pl.ds(v * 16, 16)]
                    slot = d - lo
                    msk = (slot >= 0) & (slot < PER)
                    slot = jnp.where(msk, slot, 0)
                    plsc.addupdate_scatter(acc, [slot], ones, mask=msk)
                    return c2

                lax.fori_loop(0, NV, scan, 0)
                return carry

            lax.fori_loop(0, nch, chunk, 0)

        pass_(dst_hbm, dacc)
        pass_(src_hbm, oacc)
        pltpu.sync_copy(dacc, deg_hbm.at[pl.ds(lo, PER)])
        pltpu.sync_copy(oacc, odeg_hbm.at[pl.ds(lo, PER)])

    return k(dstv, srcv)


def kernel(edge_index, etype, target_idx, edge_gid, q_emb, W_qp, b_qp,
           rel_emb, Wx, Wh, b_g, W_pna, b_pna, W_lstm, b_lstm, W_eq, b_eq,
           ln_w, ln_b, W_ejk, b_ejk, W_njk, b_njk, W_qjk, b_qjk, W_fc, b_fc):
    E = etype.shape[0]
    N = 10000
    src = edge_index[0]
    dst = edge_index[1]
    T = target_idx.shape[0]
    B = T // 2

    queries = jnp.take(q_emb, jnp.take(etype, target_idx), axis=0)
    eqm = target_idx[None, :] == target_idx[:, None]
    later = jnp.triu(eqm, 1).any(axis=1)
    tidx_masked = jnp.where(later, -1, target_idx).astype(jnp.int32)[None, :]
    T2 = queries.reshape(B, 2 * D) @ W_qp + b_qp

    src32 = src.astype(jnp.int32)
    dst32 = dst.astype(jnp.int32)
    deg_pad, odeg_pad = _sc_counts(dst32, src32)
    avg_d = jnp.mean(jnp.log(odeg_pad[:N] + 1.0)).reshape(1, 1)
    deg2 = deg_pad.reshape(-1, 1)

    equery = _sc_gather(T2, edge_gid.astype(jnp.int32))
    etype32 = etype.astype(jnp.int32)
    re0 = _sc_gather(rel_emb[0], etype32)
    re1 = _sc_gather(rel_emb[1], etype32)

    Wl1 = [W_lstm[l, :D] for l in range(2)]
    Wl2 = [W_lstm[l, D:] for l in range(2)]
    Wq1 = [W_eq[l, :D] for l in range(2)]
    Wq2 = [W_eq[l, D:2 * D] for l in range(2)]
    Wq3 = [W_eq[l, 2 * D:] for l in range(2)]
    bg = b_g.reshape(2, 1, 3 * D)
    bl = b_lstm.reshape(2, 1, 4 * D)
    bq = b_eq.reshape(2, 1, D)
    bp = b_pna.reshape(2, 1, D)
    lnw = ln_w.reshape(2, 1, D)
    lnb = ln_b.reshape(2, 1, D)

    m0 = _edge_msg_l0(re0, equery, queries, tidx_masked, Wx[0], bg[0])
    s0, mx0, mn0 = _sc_segment(m0, dst32)
    nraw0, npost0 = _pna(s0, mx0, mn0, deg2, avg_d, jnp.zeros((N, D), jnp.float32),
                         W_pna[0], bp[0], lnw[0], lnb[0])
    ns0 = _sc_gather(nraw0, src32)
    nd0 = _sc_gather(nraw0, dst32)
    ef1, eq1 = _edge_upd(True, tidx_masked, equery, ns0, nd0, queries,
                         Wl1[0], Wl2[0], bl[0], Wq1[0], Wq2[0], Wq3[0],
                         bq[0], lnw[0], lnb[0])

    h1 = _sc_gather(npost0, src32)
    m1 = _edge_msg_l1(re1, eq1, ef1, h1, Wx[1], Wh[1], bg[1])
    s1, mx1, mn1 = _sc_segment(m1, dst32)
    nraw1, npost1 = _pna(s1, mx1, mn1, deg2, avg_d, npost0,
                         W_pna[1], bp[1], lnw[1], lnb[1])
    ns1 = _sc_gather(nraw1, src32)
    nd1 = _sc_gather(nraw1, dst32)
    ef2, eq2 = _edge_upd(False, ef1, eq1, ns1, nd1, None,
                         Wl1[1], Wl2[1], bl[1], Wq1[1], Wq2[1], Wq3[1],
                         bq[1], lnw[1], lnb[1])

    tef = jnp.concatenate([jnp.take(ef1, target_idx, axis=0),
                           jnp.take(ef2, target_idx, axis=0)], axis=1)
    teq = jnp.concatenate([jnp.take(eq1, target_idx, axis=0),
                           jnp.take(eq2, target_idx, axis=0)], axis=1)
    tn = jnp.take(src, target_idx)
    tnf = jnp.concatenate([jnp.take(npost0, tn, axis=0),
                           jnp.take(npost1, tn, axis=0)], axis=1)
    u, v, w = _head(tef, teq, tnf, W_ejk, b_ejk.reshape(1, D),
                    W_njk, b_njk.reshape(1, D), W_qjk, b_qjk.reshape(1, D),
                    W_fc)
    u = u[:, 0]
    v = v[:, 0]
    w = w[:, 0]
    right = u[0::2] + v[0::2] + w[1::2] + b_fc[0]
    left = u[1::2] + v[1::2] + w[0::2] + b_fc[0]
    return jnp.maximum(right, left)[:, None]


# speedup vs baseline: 3.2928x; 3.2928x over previous
import functools

import jax
import jax.numpy as jnp
from jax import lax
from jax.experimental import pallas as pl
from jax.experimental.pallas import tpu as pltpu
from jax.experimental.pallas import tpu_sc as plsc

D = 128
_INTERPRET = False


def _block(n, candidates):
    for c in candidates:
        if n % c == 0:
            return c
    return n


def _edge_block(E):
    return _block(E, (2000, 1600, 800, 400, 200, 80, 40, 16, 8))


def _node_block(N):
    return _block(N, (400, 200, 100, 50, 40, 16, 8))


def _ln(x, w, b):
    mu = jnp.mean(x, axis=-1, keepdims=True)
    var = jnp.mean((x - mu) ** 2, axis=-1, keepdims=True)
    return (x - mu) * jax.lax.rsqrt(var + 1e-5) * w + b


def _vspec(blk, imap):
    return pl.BlockSpec(blk, imap)


def _wspec(shape):
    return pl.BlockSpec(shape, lambda *a: (0,) * len(shape))


def _edge_msg_l0_body(tidx, re, eq, tq, Wx, bg, m):
    eb = re.shape[0]
    gid = jax.lax.broadcasted_iota(jnp.int32, (eb, tidx.shape[1]), 0)
    gid = gid + pl.program_id(0) * eb
    onehot = (gid == tidx[...]).astype(jnp.float32)
    ef0 = jnp.dot(onehot, tq[...], preferred_element_type=jnp.float32)
    x = ef0 * re[...] + eq[...]
    gx = jnp.dot(x, Wx[...], preferred_element_type=jnp.float32) + bg[...]
    z = jax.nn.sigmoid(gx[:, :D])
    n = jnp.tanh(gx[:, 2 * D:])
    m[...] = z * n


def _edge_msg_l1_body(re, eq, ef, h, Wx, Wh, bg, m):
    x = ef[...] * re[...] + eq[...]
    gx = jnp.dot(x, Wx[...], preferred_element_type=jnp.float32) + bg[...]
    gh = jnp.dot(h[...], Wh[...], preferred_element_type=jnp.float32)
    z = jax.nn.sigmoid(gx[:, :D] + gh[:, :D])
    r = jax.nn.sigmoid(gx[:, D:2 * D] + gh[:, D:2 * D])
    n = jnp.tanh(gx[:, 2 * D:] + r * gh[:, 2 * D:])
    m[...] = (1.0 - z) * h[...] + z * n


def _edge_msg_l0(re, eq, tq, tidx, Wx, bg):
    E = re.shape[0]
    eb = _edge_block(E)
    grid = E // eb
    t = tidx.shape[1]
    return pl.pallas_call(
        _edge_msg_l0_body,
        grid=(grid,),
        in_specs=[
            _wspec((1, t)),
            _vspec((eb, D), lambda i: (i, 0)),
            _vspec((eb, D), lambda i: (i, 0)),
            _wspec((t, D)),
            _wspec((D, 3 * D)),
            _wspec((1, 3 * D)),
        ],
        out_specs=_vspec((eb, D), lambda i: (i, 0)),
        out_shape=jax.ShapeDtypeStruct((E, D), jnp.float32),
        interpret=_INTERPRET,
    )(tidx, re, eq, tq, Wx, bg)


def _edge_msg_l1(re, eq, ef, h, Wx, Wh, bg):
    E = re.shape[0]
    eb = _edge_block(E)
    grid = E // eb
    return pl.pallas_call(
        _edge_msg_l1_body,
        grid=(grid,),
        in_specs=[
            _vspec((eb, D), lambda i: (i, 0)),
            _vspec((eb, D), lambda i: (i, 0)),
            _vspec((eb, D), lambda i: (i, 0)),
            _vspec((eb, D), lambda i: (i, 0)),
            _wspec((D, 3 * D)),
            _wspec((D, 3 * D)),
            _wspec((1, 3 * D)),
        ],
        out_specs=_vspec((eb, D), lambda i: (i, 0)),
        out_shape=jax.ShapeDtypeStruct((E, D), jnp.float32),
        interpret=_INTERPRET,
    )(re, eq, ef, h, Wx, Wh, bg)


def _pna_body(ssum, smax, smin, degr, avgr, prev, Wp, bp, lnw, lnb, raw,
              post):
    deg = degr[...]
    degc = jnp.maximum(deg, 1.0)
    pos = deg > 0
    mean = ssum[...] / degc
    mx = jnp.where(pos, smax[...], 0.0)
    mn = jnp.where(pos, smin[...], 0.0)
    agg = jnp.concatenate([mean, mx, mn], axis=1)
    ld = jnp.log(deg + 1.0)
    avg = avgr[...]
    amp = ld / avg
    att = avg / jnp.where(ld > 0, ld, 1.0)
    h_agg = jnp.concatenate([agg, agg * amp, agg * att], axis=1)
    r = jnp.dot(h_agg, Wp[...], preferred_element_type=jnp.float32) + bp[...]
    raw[...] = r
    post[...] = prev[...] + _ln(r, lnw[...], lnb[...])


def _pna(aggs, deg, avg_d, prev, Wp, bp, lnw, lnb):
    N = prev.shape[0]
    nb = _node_block(N)
    grid = N // nb
    return pl.pallas_call(
        _pna_body,
        grid=(grid,),
        in_specs=[
            _vspec((nb, D), lambda i: (i, 0)),
            _vspec((nb, D), lambda i: (i, 0)),
            _vspec((nb, D), lambda i: (i, 0)),
            _vspec((nb, 1), lambda i: (i, 0)),
            _wspec((1, 1)),
            _vspec((nb, D), lambda i: (i, 0)),
            _wspec((9 * D, D)),
            _wspec((1, D)),
            _wspec((1, D)),
            _wspec((1, D)),
        ],
        out_specs=[
            _vspec((nb, D), lambda i: (i, 0)),
            _vspec((nb, D), lambda i: (i, 0)),
        ],
        out_shape=[
            jax.ShapeDtypeStruct((N, D), jnp.float32),
            jax.ShapeDtypeStruct((N, D), jnp.float32),
        ],
        interpret=_INTERPRET,
    )(*aggs, deg, avg_d, prev, Wp, bp, lnw, lnb)


def _edge_upd_common(ef, eq, ns, nd, Wl1, Wl2, bl, Wq1, Wq2, Wq3, bq, lnw, lnb,
                     efo, eqo):
    g4 = (jnp.dot(ns, Wl1, preferred_element_type=jnp.float32)
          + jnp.dot(nd, Wl2, preferred_element_type=jnp.float32) + bl)
    gi = g4[:, :D]
    gf = g4[:, D:2 * D]
    go = g4[:, 2 * D:3 * D]
    gc = g4[:, 3 * D:]
    c = jax.nn.sigmoid(gf) * ef + jax.nn.sigmoid(gi) * jnp.tanh(gc)
    efn = jax.nn.sigmoid(go) * jnp.tanh(c)
    eqn = jnp.tanh(jnp.dot(eq, Wq1, preferred_element_type=jnp.float32)
                   + jnp.dot(ns, Wq2, preferred_element_type=jnp.float32)
                   + jnp.dot(nd, Wq3, preferred_element_type=jnp.float32) + bq)
    efo[...] = ef + _ln(efn, lnw, lnb)
    eqo[...] = eq + _ln(eqn, lnw, lnb)


def _edge_upd_l0_body(tidx, eq, ns, nd, tq, Wl1, Wl2, bl, Wq1, Wq2, Wq3, bq,
                      lnw, lnb, efo, eqo):
    eb = eq.shape[0]
    gid = jax.lax.broadcasted_iota(jnp.int32, (eb, tidx.shape[1]), 0)
    gid = gid + pl.program_id(0) * eb
    onehot = (gid == tidx[...]).astype(jnp.float32)
    ef = jnp.dot(onehot, tq[...], preferred_element_type=jnp.float32)
    _edge_upd_common(ef, eq[...], ns[...], nd[...], Wl1[...], Wl2[...],
                     bl[...], Wq1[...], Wq2[...], Wq3[...], bq[...],
                     lnw[...], lnb[...], efo, eqo)


def _edge_upd_l1_body(ef, eq, ns, nd, Wl1, Wl2, bl, Wq1, Wq2, Wq3, bq,
                      lnw, lnb, efo, eqo):
    _edge_upd_common(ef[...], eq[...], ns[...], nd[...], Wl1[...], Wl2[...],
                     bl[...], Wq1[...], Wq2[...], Wq3[...], bq[...],
                     lnw[...], lnb[...], efo, eqo)


def _edge_upd(layer0, ef_or_tidx, eq, ns, nd, tq, Wl1, Wl2, bl, Wq1, Wq2,
              Wq3, bq, lnw, lnb):
    E = eq.shape[0]
    eb = _edge_block(E)
    grid = E // eb
    espec = _vspec((eb, D), lambda i: (i, 0))
    wspecs = [
        _wspec((D, 4 * D)), _wspec((D, 4 * D)), _wspec((1, 4 * D)),
        _wspec((D, D)), _wspec((D, D)), _wspec((D, D)), _wspec((1, D)),
        _wspec((1, D)), _wspec((1, D)),
    ]
    outs = dict(
        out_specs=[espec, espec],
        out_shape=[jax.ShapeDtypeStruct((E, D), jnp.float32),
                   jax.ShapeDtypeStruct((E, D), jnp.float32)],
        interpret=_INTERPRET,
    )
    if layer0:
        t = ef_or_tidx.shape[1]
        return pl.pallas_call(
            _edge_upd_l0_body,
            grid=(grid,),
            in_specs=[_wspec((1, t)), espec, espec, espec, _wspec((t, D))]
            + wspecs,
            **outs,
        )(ef_or_tidx, eq, ns, nd, tq, Wl1, Wl2, bl, Wq1, Wq2, Wq3, bq,
          lnw, lnb)
    return pl.pallas_call(
        _edge_upd_l1_body,
        grid=(grid,),
        in_specs=[espec, espec, espec, espec] + wspecs,
        **outs,
    )(ef_or_tidx, eq, ns, nd, Wl1, Wl2, bl, Wq1, Wq2, Wq3, bq, lnw, lnb)


def _head_body(tef, teq, tnf, Wej, bej, Wnj, bnj, Wqj, bqj, Wfc, u, v, w):
    ef32 = jnp.dot(tef[...], Wej[...], preferred_element_type=jnp.float32) + bej[...]
    eq32 = jnp.dot(teq[...], Wqj[...], preferred_element_type=jnp.float32) + bqj[...]
    nf32 = jnp.dot(tnf[...], Wnj[...], preferred_element_type=jnp.float32) + bnj[...]
    A = Wfc[:D, :]
    Bp = Wfc[D:2 * D, :]
    C = Wfc[2 * D:3 * D, :]
    Dp = Wfc[3 * D:, :]
    u[...] = (jnp.dot(ef32, A, preferred_element_type=jnp.float32)
              + jnp.dot(eq32, Bp, preferred_element_type=jnp.float32))
    v[...] = jnp.dot(nf32, C, preferred_element_type=jnp.float32)
    w[...] = jnp.dot(nf32, Dp, preferred_element_type=jnp.float32)


def _head(tef, teq, tnf, Wej, bej, Wnj, bnj, Wqj, bqj, Wfc):
    T = tef.shape[0]
    K = tef.shape[1]
    return pl.pallas_call(
        _head_body,
        in_specs=[
            _wspec((T, K)), _wspec((T, K)), _wspec((T, K)),
            _wspec((K, D)), _wspec((1, D)),
            _wspec((K, D)), _wspec((1, D)),
            _wspec((K, D)), _wspec((1, D)),
            _wspec((4 * D, 1)),
        ],
        out_specs=[_wspec((T, 1)), _wspec((T, 1)), _wspec((T, 1))],
        out_shape=[jax.ShapeDtypeStruct((T, 1), jnp.float32)] * 3,
        interpret=_INTERPRET,
    )(tef, teq, tnf, Wej, bej, Wnj, bnj, Wqj, bqj, Wfc)


def _gather_rows(table, idx):
    return jnp.take(table, idx, axis=0)


_GCH = 128


def _sc_mesh():
    return plsc.VectorSubcoreMesh(core_axis_name="c", subcore_axis_name="s")


def _sc_gather(table, idx):
    E2 = idx.shape[0]
    Dt = table.shape[1]
    assert E2 % _GCH == 0
    nch = E2 // _GCH
    info = plsc.get_sparse_core_info()
    NC, NS = info.num_cores, info.num_subcores
    NW = NC * NS
    bound = -(-nch // NW)

    @functools.partial(
        pl.kernel,
        mesh=_sc_mesh(),
        out_type=jax.ShapeDtypeStruct((E2, Dt), jnp.float32),
        scratch_types=[
            pltpu.VMEM((_GCH,), jnp.int32),
            pltpu.VMEM((_GCH, Dt), jnp.float32),
            pltpu.SemaphoreType.DMA,
        ],
    )
    def k(table_hbm, idx_hbm, out_hbm, idxb, rows, sem):
        wid = lax.axis_index("s") * NC + lax.axis_index("c")

        def body(j, carry):
            c = wid + j * NW

            @pl.when(c < nch)
            def _():
                base = c * _GCH
                pltpu.sync_copy(idx_hbm.at[pl.ds(base, _GCH)], idxb)
                pltpu.async_copy(table_hbm.at[idxb], rows, sem).wait()
                pltpu.sync_copy(rows, out_hbm.at[pl.ds(base, _GCH)])

            return carry

        lax.fori_loop(0, bound, body, 0)

    return k(table, idx)


def _segment_reduce(m, dst, N):
    ssum = jax.ops.segment_sum(m, dst, num_segments=N)
    smax = jax.ops.segment_max(m, dst, num_segments=N)
    smin = jax.ops.segment_min(m, dst, num_segments=N)
    return ssum, smax, smin


_SEG_PER = 160
_SEG_NP = 2
_SEG_CHD = 8000
_SEG_RCH = 128


def _sc_segment(m, dstv):
    E2 = dstv.shape[0]
    Dt = m.shape[1]
    info = plsc.get_sparse_core_info()
    NC = info.num_cores
    NW = NC * info.num_subcores
    PER = _SEG_PER
    Npad = _SEG_NP * NW * PER
    nch = E2 // _SEG_CHD
    NV = _SEG_CHD // 16
    NSUB = -(-_SEG_CHD // _SEG_RCH)
    NGRP = _SEG_RCH // 16
    FMAX = 3.4028235e38

    @functools.partial(
        pl.kernel,
        mesh=_sc_mesh(),
        compiler_params=pltpu.CompilerParams(needs_layout_passes=False),
        out_type=(jax.ShapeDtypeStruct((Npad, Dt), jnp.float32),
                  jax.ShapeDtypeStruct((Npad, Dt), jnp.float32),
                  jax.ShapeDtypeStruct((Npad, Dt), jnp.float32)),
        scratch_types=[
            pltpu.VMEM((_SEG_CHD,), jnp.int32),
            pltpu.VMEM((_SEG_CHD + _SEG_RCH,), jnp.int32),
            pltpu.VMEM((_SEG_CHD + 16,), jnp.int32),
            pltpu.VMEM((_SEG_RCH, Dt), jnp.float32),
            pltpu.VMEM((PER + 1, Dt), jnp.float32),
            pltpu.VMEM((PER + 1, Dt), jnp.float32),
            pltpu.VMEM((PER + 1, Dt), jnp.float32),
            pltpu.SemaphoreType.DMA,
        ],
    )
    def k(m_hbm, dst_hbm, sum_hbm, max_hbm, min_hbm,
          dstb, idxb, slotb, rows, ssum, smax, smin, sem):
        wid = lax.axis_index("s") * NC + lax.axis_index("c")
        zero16 = jnp.zeros((16,), jnp.float32)
        ninf = jnp.full((16,), -FMAX, jnp.float32)
        pinf = jnp.full((16,), FMAX, jnp.float32)
        izero = jnp.zeros((16,), jnp.int32)
        iota = lax.iota(jnp.int32, 16)

        def init_idx(i, c):
            idxb[pl.ds(i * 16, 16)] = izero
            return c

        lax.fori_loop(0, (_SEG_CHD + _SEG_RCH) // 16, init_idx, 0)

        for p in range(_SEG_NP):
            lo = (p * NW + wid) * PER

            def init_row(i, c):
                for kq in range(Dt // 16):
                    ssum[i, pl.ds(kq * 16, 16)] = zero16
                    smax[i, pl.ds(kq * 16, 16)] = ninf
                    smin[i, pl.ds(kq * 16, 16)] = pinf
                return c

            lax.fori_loop(0, PER + 1, init_row, 0)

            def chunk(c, carry):
                base = c * _SEG_CHD
                pltpu.sync_copy(dst_hbm.at[pl.ds(base, _SEG_CHD)], dstb)

                def scan(v, cnt):
                    d = dstb[pl.ds(v * 16, 16)]
                    slot = d - lo
                    msk = (slot >= 0) & (slot < PER)
                    eidx = base + v * 16 + iota
                    plsc.store_compressed(idxb.at[pl.ds(cnt, 16)], eidx,
                                          mask=msk)
                    plsc.store_compressed(slotb.at[pl.ds(cnt, 16)], slot,
                                          mask=msk)
                    return cnt + jnp.sum(msk.astype(jnp.int32))

                total = lax.fori_loop(0, NV, scan, 0)

                def sub(g, c2):
                    gb = g * _SEG_RCH

                    @pl.when(gb < total)
                    def _():
                        pltpu.async_copy(
                            m_hbm.at[idxb.at[pl.ds(gb, _SEG_RCH)]], rows, sem
                        ).wait()

                        def grp(g16, c3):
                            jb = g16 * 16
                            rem = total - gb - jb
                            sv = slotb[pl.ds(gb + jb, 16)]
                            sv = jnp.where(iota < rem, sv, PER)
                            for i in range(16):
                                slot = sv[i]
                                for kq in range(Dt // 16):
                                    dsl = pl.ds(kq * 16, 16)
                                    rv = rows[jb + i, dsl]
                                    ssum[slot, dsl] = ssum[slot, dsl] + rv
                                    smax[slot, dsl] = jnp.maximum(
                                        smax[slot, dsl], rv)
                                    smin[slot, dsl] = jnp.minimum(
                                        smin[slot, dsl], rv)
                            return c3

                        lax.fori_loop(0, NGRP, grp, 0)

                    return c2

                lax.fori_loop(0, NSUB, sub, 0)
                return carry

            lax.fori_loop(0, nch, chunk, 0)

            pltpu.sync_copy(ssum.at[pl.ds(0, PER)], sum_hbm.at[pl.ds(lo, PER)])
            pltpu.sync_copy(smax.at[pl.ds(0, PER)], max_hbm.at[pl.ds(lo, PER)])
            pltpu.sync_copy(smin.at[pl.ds(0, PER)], min_hbm.at[pl.ds(lo, PER)])

    return k(m, dstv)


def _sc_counts(dstv, srcv):
    E2 = dstv.shape[0]
    info = plsc.get_sparse_core_info()
    NC = info.num_cores
    NW = NC * info.num_subcores
    PER = 320
    Npad = NW * PER
    nch = E2 // _SEG_CHD
    NV = _SEG_CHD // 16

    @functools.partial(
        pl.kernel,
        mesh=_sc_mesh(),
        compiler_params=pltpu.CompilerParams(needs_layout_passes=False),
        out_type=(jax.ShapeDtypeStruct((Npad,), jnp.float32),
                  jax.ShapeDtypeStruct((Npad,), jnp.float32)),
        scratch_types=[
            pltpu.VMEM((_SEG_CHD,), jnp.int32),
            pltpu.VMEM((PER,), jnp.float32),
            pltpu.VMEM((PER,), jnp.float32),
            pltpu.SemaphoreType.DMA,
        ],
    )
    def k(dst_hbm, src_hbm, deg_hbm, odeg_hbm, idxc, dacc, oacc, sem):
        wid = lax.axis_index("s") * NC + lax.axis_index("c")
        lo = wid * PER
        zero16 = jnp.zeros((16,), jnp.float32)
        ones = jnp.ones((16,), jnp.float32)

        def init(i, c):
            dacc[pl.ds(i * 16, 16)] = zero16
            oacc[pl.ds(i * 16, 16)] = zero16
            return c

        lax.fori_loop(0, PER // 16, init, 0)

        def pass_(idx_hbm, acc):
            def chunk(c, carry):
                pltpu.sync_copy(idx_hbm.at[pl.ds(c * _SEG_CHD, _SEG_CHD)], idxc)

                def scan(v, c2):
                    d = idxc[pl.ds(v * 16, 16)]
                    slot = d - lo
                    msk = (slot >= 0) & (slot < PER)
                    slot = jnp.where(msk, slot, 0)
                    plsc.addupdate_scatter(acc, [slot], ones, mask=msk)
                    return c2

                lax.fori_loop(0, NV, scan, 0)
                return carry

            lax.fori_loop(0, nch, chunk, 0)

        pass_(dst_hbm, dacc)
        pass_(src_hbm, oacc)
        pltpu.sync_copy(dacc, deg_hbm.at[pl.ds(lo, PER)])
        pltpu.sync_copy(oacc, odeg_hbm.at[pl.ds(lo, PER)])

    return k(dstv, srcv)


def kernel(edge_index, etype, target_idx, edge_gid, q_emb, W_qp, b_qp,
           rel_emb, Wx, Wh, b_g, W_pna, b_pna, W_lstm, b_lstm, W_eq, b_eq,
           ln_w, ln_b, W_ejk, b_ejk, W_njk, b_njk, W_qjk, b_qjk, W_fc, b_fc):
    E = etype.shape[0]
    N = 10000
    src = edge_index[0]
    dst = edge_index[1]
    T = target_idx.shape[0]
    B = T // 2

    queries = jnp.take(q_emb, jnp.take(etype, target_idx), axis=0)
    eqm = target_idx[None, :] == target_idx[:, None]
    later = jnp.triu(eqm, 1).any(axis=1)
    tidx_masked = jnp.where(later, -1, target_idx).astype(jnp.int32)[None, :]
    T2 = queries.reshape(B, 2 * D) @ W_qp + b_qp

    src32 = src.astype(jnp.int32)
    dst32 = dst.astype(jnp.int32)
    deg_pad, odeg_pad = _sc_counts(dst32, src32)
    avg_d = jnp.mean(jnp.log(odeg_pad[:N] + 1.0)).reshape(1, 1)
    deg2 = deg_pad.reshape(-1, 1)

    equery = _sc_gather(T2, edge_gid.astype(jnp.int32))
    etype32 = etype.astype(jnp.int32)
    re0 = _sc_gather(rel_emb[0], etype32)
    re1 = _sc_gather(rel_emb[1], etype32)

    Wl1 = [W_lstm[l, :D] for l in range(2)]
    Wl2 = [W_lstm[l, D:] for l in range(2)]
    Wq1 = [W_eq[l, :D] for l in range(2)]
    Wq2 = [W_eq[l, D:2 * D] for l in range(2)]
    Wq3 = [W_eq[l, 2 * D:] for l in range(2)]
    bg = b_g.reshape(2, 1, 3 * D)
    bl = b_lstm.reshape(2, 1, 4 * D)
    bq = b_eq.reshape(2, 1, D)
    bp = b_pna.reshape(2, 1, D)
    lnw = ln_w.reshape(2, 1, D)
    lnb = ln_b.reshape(2, 1, D)

    m0 = _edge_msg_l0(re0, equery, queries, tidx_masked, Wx[0], bg[0])
    s0, mx0, mn0 = _sc_segment(m0, dst32)
    nraw0, npost0 = _pna((s0, mx0, mn0), deg2, avg_d,
                         jnp.zeros((N, D), jnp.float32),
                         W_pna[0], bp[0], lnw[0], lnb[0])
    ns0 = _sc_gather(nraw0, src32)
    nd0 = _sc_gather(nraw0, dst32)
    ef1, eq1 = _edge_upd(True, tidx_masked, equery, ns0, nd0, queries,
                         Wl1[0], Wl2[0], bl[0], Wq1[0], Wq2[0], Wq3[0],
                         bq[0], lnw[0], lnb[0])

    h1 = _sc_gather(npost0, src32)
    m1 = _edge_msg_l1(re1, eq1, ef1, h1, Wx[1], Wh[1], bg[1])
    s1, mx1, mn1 = _sc_segment(m1, dst32)
    nraw1, npost1 = _pna((s1, mx1, mn1), deg2, avg_d,
                         npost0, W_pna[1], bp[1], lnw[1], lnb[1])
    ns1 = _sc_gather(nraw1, src32)
    nd1 = _sc_gather(nraw1, dst32)
    ef2, eq2 = _edge_upd(False, ef1, eq1, ns1, nd1, None,
                         Wl1[1], Wl2[1], bl[1], Wq1[1], Wq2[1], Wq3[1],
                         bq[1], lnw[1], lnb[1])

    tef = jnp.concatenate([jnp.take(ef1, target_idx, axis=0),
                           jnp.take(ef2, target_idx, axis=0)], axis=1)
    teq = jnp.concatenate([jnp.take(eq1, target_idx, axis=0),
                           jnp.take(eq2, target_idx, axis=0)], axis=1)
    tn = jnp.take(src, target_idx)
    tnf = jnp.concatenate([jnp.take(npost0, tn, axis=0),
                           jnp.take(npost1, tn, axis=0)], axis=1)
    u, v, w = _head(tef, teq, tnf, W_ejk, b_ejk.reshape(1, D),
                    W_njk, b_njk.reshape(1, D), W_qjk, b_qjk.reshape(1, D),
                    W_fc)
    u = u[:, 0]
    v = v[:, 0]
    w = w[:, 0]
    right = u[0::2] + v[0::2] + w[1::2] + b_fc[0]
    left = u[1::2] + v[1::2] + w[0::2] + b_fc[0]
    return jnp.maximum(right, left)[:, None]


# speedup vs baseline: 4.8706x; 1.4792x over previous
import functools

import jax
import jax.numpy as jnp
from jax import lax
from jax.experimental import pallas as pl
from jax.experimental.pallas import tpu as pltpu
from jax.experimental.pallas import tpu_sc as plsc

D = 128
_INTERPRET = False


def _block(n, candidates):
    for c in candidates:
        if n % c == 0:
            return c
    return n


def _edge_block(E):
    return _block(E, (2000, 1600, 800, 400, 200, 80, 40, 16, 8))


def _node_block(N):
    return _block(N, (400, 200, 100, 50, 40, 16, 8))


def _ln(x, w, b):
    mu = jnp.mean(x, axis=-1, keepdims=True)
    var = jnp.mean((x - mu) ** 2, axis=-1, keepdims=True)
    return (x - mu) * jax.lax.rsqrt(var + 1e-5) * w + b


def _vspec(blk, imap):
    return pl.BlockSpec(blk, imap)


def _wspec(shape):
    return pl.BlockSpec(shape, lambda *a: (0,) * len(shape))


def _edge_msg_l0_body(tidx, re, eq, tq, Wx, bg, m):
    eb = re.shape[0]
    gid = jax.lax.broadcasted_iota(jnp.int32, (eb, tidx.shape[1]), 0)
    gid = gid + pl.program_id(0) * eb
    onehot = (gid == tidx[...]).astype(jnp.float32)
    ef0 = jnp.dot(onehot, tq[...], preferred_element_type=jnp.float32)
    x = ef0 * re[...] + eq[...]
    gx = jnp.dot(x, Wx[...], preferred_element_type=jnp.float32) + bg[...]
    z = jax.nn.sigmoid(gx[:, :D])
    n = jnp.tanh(gx[:, 2 * D:])
    m[...] = z * n


def _edge_msg_l1_body(re, eq, ef, h, Wx, Wh, bg, m):
    x = ef[...] * re[...] + eq[...]
    gx = jnp.dot(x, Wx[...], preferred_element_type=jnp.float32) + bg[...]
    gh = jnp.dot(h[...], Wh[...], preferred_element_type=jnp.float32)
    z = jax.nn.sigmoid(gx[:, :D] + gh[:, :D])
    r = jax.nn.sigmoid(gx[:, D:2 * D] + gh[:, D:2 * D])
    n = jnp.tanh(gx[:, 2 * D:] + r * gh[:, 2 * D:])
    m[...] = (1.0 - z) * h[...] + z * n


def _edge_msg_l0(re, eq, tq, tidx, Wx, bg):
    E = re.shape[0]
    eb = _edge_block(E)
    grid = E // eb
    t = tidx.shape[1]
    return pl.pallas_call(
        _edge_msg_l0_body,
        grid=(grid,),
        in_specs=[
            _wspec((1, t)),
            _vspec((eb, D), lambda i: (i, 0)),
            _vspec((eb, D), lambda i: (i, 0)),
            _wspec((t, D)),
            _wspec((D, 3 * D)),
            _wspec((1, 3 * D)),
        ],
        out_specs=_vspec((eb, D), lambda i: (i, 0)),
        out_shape=jax.ShapeDtypeStruct((E, D), jnp.float32),
        interpret=_INTERPRET,
    )(tidx, re, eq, tq, Wx, bg)


def _edge_msg_l1(re, eq, ef, h, Wx, Wh, bg):
    E = re.shape[0]
    eb = _edge_block(E)
    grid = E // eb
    return pl.pallas_call(
        _edge_msg_l1_body,
        grid=(grid,),
        in_specs=[
            _vspec((eb, D), lambda i: (i, 0)),
            _vspec((eb, D), lambda i: (i, 0)),
            _vspec((eb, D), lambda i: (i, 0)),
            _vspec((eb, D), lambda i: (i, 0)),
            _wspec((D, 3 * D)),
            _wspec((D, 3 * D)),
            _wspec((1, 3 * D)),
        ],
        out_specs=_vspec((eb, D), lambda i: (i, 0)),
        out_shape=jax.ShapeDtypeStruct((E, D), jnp.float32),
        interpret=_INTERPRET,
    )(re, eq, ef, h, Wx, Wh, bg)


def _pna_body(ssum, smax, smin, degr, avgr, prev, Wp, bp, lnw, lnb, raw,
              post):
    deg = degr[...]
    degc = jnp.maximum(deg, 1.0)
    pos = deg > 0
    mean = ssum[...] / degc
    mx = jnp.where(pos, smax[...], 0.0)
    mn = jnp.where(pos, smin[...], 0.0)
    agg = jnp.concatenate([mean, mx, mn], axis=1)
    ld = jnp.log(deg + 1.0)
    avg = avgr[...]
    amp = ld / avg
    att = avg / jnp.where(ld > 0, ld, 1.0)
    h_agg = jnp.concatenate([agg, agg * amp, agg * att], axis=1)
    r = jnp.dot(h_agg, Wp[...], preferred_element_type=jnp.float32) + bp[...]
    raw[...] = r
    post[...] = prev[...] + _ln(r, lnw[...], lnb[...])


def _pna(aggs, deg, avg_d, prev, Wp, bp, lnw, lnb):
    N = prev.shape[0]
    nb = _node_block(N)
    grid = N // nb
    return pl.pallas_call(
        _pna_body,
        grid=(grid,),
        in_specs=[
            _vspec((nb, D), lambda i: (i, 0)),
            _vspec((nb, D), lambda i: (i, 0)),
            _vspec((nb, D), lambda i: (i, 0)),
            _vspec((nb, 1), lambda i: (i, 0)),
            _wspec((1, 1)),
            _vspec((nb, D), lambda i: (i, 0)),
            _wspec((9 * D, D)),
            _wspec((1, D)),
            _wspec((1, D)),
            _wspec((1, D)),
        ],
        out_specs=[
            _vspec((nb, D), lambda i: (i, 0)),
            _vspec((nb, D), lambda i: (i, 0)),
        ],
        out_shape=[
            jax.ShapeDtypeStruct((N, D), jnp.float32),
            jax.ShapeDtypeStruct((N, D), jnp.float32),
        ],
        interpret=_INTERPRET,
    )(*aggs, deg, avg_d, prev, Wp, bp, lnw, lnb)


def _edge_upd_common(ef, eq, ns, nd, Wl1, Wl2, bl, Wq1, Wq2, Wq3, bq, lnw, lnb,
                     efo, eqo):
    g4 = (jnp.dot(ns, Wl1, preferred_element_type=jnp.float32)
          + jnp.dot(nd, Wl2, preferred_element_type=jnp.float32) + bl)
    gi = g4[:, :D]
    gf = g4[:, D:2 * D]
    go = g4[:, 2 * D:3 * D]
    gc = g4[:, 3 * D:]
    c = jax.nn.sigmoid(gf) * ef + jax.nn.sigmoid(gi) * jnp.tanh(gc)
    efn = jax.nn.sigmoid(go) * jnp.tanh(c)
    eqn = jnp.tanh(jnp.dot(eq, Wq1, preferred_element_type=jnp.float32)
                   + jnp.dot(ns, Wq2, preferred_element_type=jnp.float32)
                   + jnp.dot(nd, Wq3, preferred_element_type=jnp.float32) + bq)
    efo[...] = ef + _ln(efn, lnw, lnb)
    eqo[...] = eq + _ln(eqn, lnw, lnb)


def _edge_upd_l0_body(tidx, eq, ns, nd, tq, Wl1, Wl2, bl, Wq1, Wq2, Wq3, bq,
                      lnw, lnb, efo, eqo):
    eb = eq.shape[0]
    gid = jax.lax.broadcasted_iota(jnp.int32, (eb, tidx.shape[1]), 0)
    gid = gid + pl.program_id(0) * eb
    onehot = (gid == tidx[...]).astype(jnp.float32)
    ef = jnp.dot(onehot, tq[...], preferred_element_type=jnp.float32)
    _edge_upd_common(ef, eq[...], ns[...], nd[...], Wl1[...], Wl2[...],
                     bl[...], Wq1[...], Wq2[...], Wq3[...], bq[...],
                     lnw[...], lnb[...], efo, eqo)


def _edge_upd_l1_body(ef, eq, ns, nd, Wl1, Wl2, bl, Wq1, Wq2, Wq3, bq,
                      lnw, lnb, efo, eqo):
    _edge_upd_common(ef[...], eq[...], ns[...], nd[...], Wl1[...], Wl2[...],
                     bl[...], Wq1[...], Wq2[...], Wq3[...], bq[...],
                     lnw[...], lnb[...], efo, eqo)


def _edge_upd(layer0, ef_or_tidx, eq, ns, nd, tq, Wl1, Wl2, bl, Wq1, Wq2,
              Wq3, bq, lnw, lnb):
    E = eq.shape[0]
    eb = _edge_block(E)
    grid = E // eb
    espec = _vspec((eb, D), lambda i: (i, 0))
    wspecs = [
        _wspec((D, 4 * D)), _wspec((D, 4 * D)), _wspec((1, 4 * D)),
        _wspec((D, D)), _wspec((D, D)), _wspec((D, D)), _wspec((1, D)),
        _wspec((1, D)), _wspec((1, D)),
    ]
    outs = dict(
        out_specs=[espec, espec],
        out_shape=[jax.ShapeDtypeStruct((E, D), jnp.float32),
                   jax.ShapeDtypeStruct((E, D), jnp.float32)],
        interpret=_INTERPRET,
    )
    if layer0:
        t = ef_or_tidx.shape[1]
        return pl.pallas_call(
            _edge_upd_l0_body,
            grid=(grid,),
            in_specs=[_wspec((1, t)), espec, espec, espec, _wspec((t, D))]
            + wspecs,
            **outs,
        )(ef_or_tidx, eq, ns, nd, tq, Wl1, Wl2, bl, Wq1, Wq2, Wq3, bq,
          lnw, lnb)
    return pl.pallas_call(
        _edge_upd_l1_body,
        grid=(grid,),
        in_specs=[espec, espec, espec, espec] + wspecs,
        **outs,
    )(ef_or_tidx, eq, ns, nd, Wl1, Wl2, bl, Wq1, Wq2, Wq3, bq, lnw, lnb)


def _head_body(tef, teq, tnf, Wej, bej, Wnj, bnj, Wqj, bqj, Wfc, u, v, w):
    ef32 = jnp.dot(tef[...], Wej[...], preferred_element_type=jnp.float32) + bej[...]
    eq32 = jnp.dot(teq[...], Wqj[...], preferred_element_type=jnp.float32) + bqj[...]
    nf32 = jnp.dot(tnf[...], Wnj[...], preferred_element_type=jnp.float32) + bnj[...]
    A = Wfc[:D, :]
    Bp = Wfc[D:2 * D, :]
    C = Wfc[2 * D:3 * D, :]
    Dp = Wfc[3 * D:, :]
    u[...] = (jnp.dot(ef32, A, preferred_element_type=jnp.float32)
              + jnp.dot(eq32, Bp, preferred_element_type=jnp.float32))
    v[...] = jnp.dot(nf32, C, preferred_element_type=jnp.float32)
    w[...] = jnp.dot(nf32, Dp, preferred_element_type=jnp.float32)


def _head(tef, teq, tnf, Wej, bej, Wnj, bnj, Wqj, bqj, Wfc):
    T = tef.shape[0]
    K = tef.shape[1]
    return pl.pallas_call(
        _head_body,
        in_specs=[
            _wspec((T, K)), _wspec((T, K)), _wspec((T, K)),
            _wspec((K, D)), _wspec((1, D)),
            _wspec((K, D)), _wspec((1, D)),
            _wspec((K, D)), _wspec((1, D)),
            _wspec((4 * D, 1)),
        ],
        out_specs=[_wspec((T, 1)), _wspec((T, 1)), _wspec((T, 1))],
        out_shape=[jax.ShapeDtypeStruct((T, 1), jnp.float32)] * 3,
        interpret=_INTERPRET,
    )(tef, teq, tnf, Wej, bej, Wnj, bnj, Wqj, bqj, Wfc)


def _gather_rows(table, idx):
    return jnp.take(table, idx, axis=0)


_GCH = 128


def _sc_mesh():
    return plsc.VectorSubcoreMesh(core_axis_name="c", subcore_axis_name="s")


def _sc_gather(table, idx):
    E2 = idx.shape[0]
    Dt = table.shape[1]
    assert E2 % _GCH == 0
    nch = E2 // _GCH
    info = plsc.get_sparse_core_info()
    NC, NS = info.num_cores, info.num_subcores
    NW = NC * NS
    bound = -(-nch // NW)

    @functools.partial(
        pl.kernel,
        mesh=_sc_mesh(),
        out_type=jax.ShapeDtypeStruct((E2, Dt), jnp.float32),
        scratch_types=[
            pltpu.VMEM((_GCH,), jnp.int32),
            pltpu.VMEM((_GCH, Dt), jnp.float32),
            pltpu.SemaphoreType.DMA,
        ],
    )
    def k(table_hbm, idx_hbm, out_hbm, idxb, rows, sem):
        wid = lax.axis_index("s") * NC + lax.axis_index("c")

        def body(j, carry):
            c = wid + j * NW

            @pl.when(c < nch)
            def _():
                base = c * _GCH
                pltpu.sync_copy(idx_hbm.at[pl.ds(base, _GCH)], idxb)
                pltpu.async_copy(table_hbm.at[idxb], rows, sem).wait()
                pltpu.sync_copy(rows, out_hbm.at[pl.ds(base, _GCH)])

            return carry

        lax.fori_loop(0, bound, body, 0)

    return k(table, idx)


def _segment_reduce(m, dst, N):
    ssum = jax.ops.segment_sum(m, dst, num_segments=N)
    smax = jax.ops.segment_max(m, dst, num_segments=N)
    smin = jax.ops.segment_min(m, dst, num_segments=N)
    return ssum, smax, smin


_SEG_PER = 160
_SEG_NP = 2
_SEG_CHD = 8000
_SEG_RCH = 128


def _sc_segment(m, dstv):
    E2 = dstv.shape[0]
    Dt = m.shape[1]
    info = plsc.get_sparse_core_info()
    NC = info.num_cores
    NW = NC * info.num_subcores
    PER = _SEG_PER
    Npad = _SEG_NP * NW * PER
    nch = E2 // _SEG_CHD
    NV = _SEG_CHD // 16
    NSUB = -(-_SEG_CHD // _SEG_RCH)
    NGRP = _SEG_RCH // 16
    FMAX = 3.4028235e38

    @functools.partial(
        pl.kernel,
        mesh=_sc_mesh(),
        compiler_params=pltpu.CompilerParams(needs_layout_passes=False),
        out_type=(jax.ShapeDtypeStruct((Npad, Dt), jnp.float32),
                  jax.ShapeDtypeStruct((Npad, Dt), jnp.float32),
                  jax.ShapeDtypeStruct((Npad, Dt), jnp.float32)),
        scratch_types=[
            pltpu.VMEM((_SEG_CHD,), jnp.int32),
            pltpu.VMEM((_SEG_CHD + _SEG_RCH,), jnp.int32),
            pltpu.VMEM((_SEG_CHD + 16,), jnp.int32),
            pltpu.VMEM((_SEG_RCH, Dt), jnp.float32),
            pltpu.VMEM((PER + 1, Dt), jnp.float32),
            pltpu.VMEM((PER + 1, Dt), jnp.float32),
            pltpu.VMEM((PER + 1, Dt), jnp.float32),
            pltpu.SemaphoreType.DMA,
        ],
    )
    def k(m_hbm, dst_hbm, sum_hbm, max_hbm, min_hbm,
          dstb, idxb, slotb, rows, ssum, smax, smin, sem):
        wid = lax.axis_index("s") * NC + lax.axis_index("c")
        zero16 = jnp.zeros((16,), jnp.float32)
        ninf = jnp.full((16,), -FMAX, jnp.float32)
        pinf = jnp.full((16,), FMAX, jnp.float32)
        izero = jnp.zeros((16,), jnp.int32)
        iota = lax.iota(jnp.int32, 16)

        def init_idx(i, c):
            idxb[pl.ds(i * 16, 16)] = izero
            return c

        lax.fori_loop(0, (_SEG_CHD + _SEG_RCH) // 16, init_idx, 0)

        for p in range(_SEG_NP):
            lo = (p * NW + wid) * PER

            def init_row(i, c):
                for kq in range(Dt // 16):
                    ssum[i, pl.ds(kq * 16, 16)] = zero16
                    smax[i, pl.ds(kq * 16, 16)] = ninf
                    smin[i, pl.ds(kq * 16, 16)] = pinf
                return c

            lax.fori_loop(0, PER + 1, init_row, 0)

            def chunk(c, carry):
                base = c * _SEG_CHD
                pltpu.sync_copy(dst_hbm.at[pl.ds(base, _SEG_CHD)], dstb)

                def scan(v, cnt):
                    d = dstb[pl.ds(v * 16, 16)]
                    slot = d - lo
                    msk = (slot >= 0) & (slot < PER)
                    eidx = base + v * 16 + iota
                    plsc.store_compressed(idxb.at[pl.ds(cnt, 16)], eidx,
                                          mask=msk)
                    plsc.store_compressed(slotb.at[pl.ds(cnt, 16)], slot,
                                          mask=msk)
                    return cnt + jnp.sum(msk.astype(jnp.int32))

                total = lax.fori_loop(0, NV, scan, 0)

                def sub(g, c2):
                    gb = g * _SEG_RCH

                    @pl.when(gb < total)
                    def _():
                        pltpu.async_copy(
                            m_hbm.at[idxb.at[pl.ds(gb, _SEG_RCH)]], rows, sem
                        ).wait()

                        def grp(g16, c3):
                            jb = g16 * 16
                            rem = total - gb - jb
                            sv = slotb[pl.ds(gb + jb, 16)]
                            sv = jnp.where(iota < rem, sv, PER)
                            for i in range(16):
                                slot = sv[i]
                                for kq in range(Dt // 16):
                                    dsl = pl.ds(kq * 16, 16)
                                    rv = rows[jb + i, dsl]
                                    ssum[slot, dsl] = ssum[slot, dsl] + rv
                                    smax[slot, dsl] = jnp.maximum(
                                        smax[slot, dsl], rv)
                                    smin[slot, dsl] = jnp.minimum(
                                        smin[slot, dsl], rv)
                            return c3

                        lax.fori_loop(0, NGRP, grp, 0)

                    return c2

                lax.fori_loop(0, NSUB, sub, 0)
                return carry

            lax.fori_loop(0, nch, chunk, 0)

            pltpu.sync_copy(ssum.at[pl.ds(0, PER)], sum_hbm.at[pl.ds(lo, PER)])
            pltpu.sync_copy(smax.at[pl.ds(0, PER)], max_hbm.at[pl.ds(lo, PER)])
            pltpu.sync_copy(smin.at[pl.ds(0, PER)], min_hbm.at[pl.ds(lo, PER)])

    return k(m, dstv)


def _sc_counts(dstv, srcv):
    E2 = dstv.shape[0]
    info = plsc.get_sparse_core_info()
    NC = info.num_cores
    NW = NC * info.num_subcores
    PER = 320
    Npad = NW * PER
    nch = E2 // _SEG_CHD
    NV = _SEG_CHD // 16

    @functools.partial(
        pl.kernel,
        mesh=_sc_mesh(),
        compiler_params=pltpu.CompilerParams(needs_layout_passes=False),
        out_type=(jax.ShapeDtypeStruct((Npad,), jnp.float32),
                  jax.ShapeDtypeStruct((Npad,), jnp.float32)),
        scratch_types=[
            pltpu.VMEM((_SEG_CHD,), jnp.int32),
            pltpu.VMEM((PER,), jnp.float32),
            pltpu.VMEM((PER,), jnp.float32),
            pltpu.SemaphoreType.DMA,
        ],
    )
    def k(dst_hbm, src_hbm, deg_hbm, odeg_hbm, idxc, dacc, oacc, sem):
        wid = lax.axis_index("s") * NC + lax.axis_index("c")
        lo = wid * PER
        zero16 = jnp.zeros((16,), jnp.float32)
        ones = jnp.ones((16,), jnp.float32)

        def init(i, c):
            dacc[pl.ds(i * 16, 16)] = zero16
            oacc[pl.ds(i * 16, 16)] = zero16
            return c

        lax.fori_loop(0, PER // 16, init, 0)

        def pass_(idx_hbm, acc):
            def chunk(c, carry):
                pltpu.sync_copy(idx_hbm.at[pl.ds(c * _SEG_CHD, _SEG_CHD)], idxc)

                def scan(v, c2):
                    d = idxc[pl.ds(v * 16, 16)]
                    slot = d - lo
                    msk = (slot >= 0) & (slot < PER)
                    slot = jnp.where(msk, slot, 0)
                    plsc.addupdate_scatter(acc, [slot], ones, mask=msk)
                    return c2

                lax.fori_loop(0, NV, scan, 0)
                return carry

            lax.fori_loop(0, nch, chunk, 0)

        pass_(dst_hbm, dacc)
        pass_(src_hbm, oacc)
        pltpu.sync_copy(dacc, deg_hbm.at[pl.ds(lo, PER)])
        pltpu.sync_copy(oacc, odeg_hbm.at[pl.ds(lo, PER)])

    return k(dstv, srcv)


def kernel(edge_index, etype, target_idx, edge_gid, q_emb, W_qp, b_qp,
           rel_emb, Wx, Wh, b_g, W_pna, b_pna, W_lstm, b_lstm, W_eq, b_eq,
           ln_w, ln_b, W_ejk, b_ejk, W_njk, b_njk, W_qjk, b_qjk, W_fc, b_fc):
    E = etype.shape[0]
    N = 10000
    src = edge_index[0]
    dst = edge_index[1]
    T = target_idx.shape[0]
    B = T // 2

    queries = jnp.take(q_emb, jnp.take(etype, target_idx), axis=0)
    eqm = target_idx[None, :] == target_idx[:, None]
    later = jnp.triu(eqm, 1).any(axis=1)
    tidx_masked = jnp.where(later, -1, target_idx).astype(jnp.int32)[None, :]
    T2 = queries.reshape(B, 2 * D) @ W_qp + b_qp

    src32 = src.astype(jnp.int32)
    dst32 = dst.astype(jnp.int32)
    ones_e = jnp.ones((E,), jnp.float32)
    out_deg = jax.ops.segment_sum(ones_e, src, num_segments=N)
    deg = jax.ops.segment_sum(ones_e, dst, num_segments=N)
    avg_d = jnp.mean(jnp.log(out_deg + 1.0)).reshape(1, 1)
    deg2 = deg.reshape(N, 1)

    equery = _sc_gather(T2, edge_gid.astype(jnp.int32))
    etype32 = etype.astype(jnp.int32)
    re0 = _sc_gather(rel_emb[0], etype32)
    re1 = _sc_gather(rel_emb[1], etype32)

    Wl1 = [W_lstm[l, :D] for l in range(2)]
    Wl2 = [W_lstm[l, D:] for l in range(2)]
    Wq1 = [W_eq[l, :D] for l in range(2)]
    Wq2 = [W_eq[l, D:2 * D] for l in range(2)]
    Wq3 = [W_eq[l, 2 * D:] for l in range(2)]
    bg = b_g.reshape(2, 1, 3 * D)
    bl = b_lstm.reshape(2, 1, 4 * D)
    bq = b_eq.reshape(2, 1, D)
    bp = b_pna.reshape(2, 1, D)
    lnw = ln_w.reshape(2, 1, D)
    lnb = ln_b.reshape(2, 1, D)

    m0 = _edge_msg_l0(re0, equery, queries, tidx_masked, Wx[0], bg[0])
    s0, mx0, mn0 = _segment_reduce(m0, dst, N)
    nraw0, npost0 = _pna((s0, mx0, mn0), deg2, avg_d,
                         jnp.zeros((N, D), jnp.float32),
                         W_pna[0], bp[0], lnw[0], lnb[0])
    ns0 = _sc_gather(nraw0, src32)
    nd0 = _sc_gather(nraw0, dst32)
    ef1, eq1 = _edge_upd(True, tidx_masked, equery, ns0, nd0, queries,
                         Wl1[0], Wl2[0], bl[0], Wq1[0], Wq2[0], Wq3[0],
                         bq[0], lnw[0], lnb[0])

    h1 = _sc_gather(npost0, src32)
    m1 = _edge_msg_l1(re1, eq1, ef1, h1, Wx[1], Wh[1], bg[1])
    s1, mx1, mn1 = _segment_reduce(m1, dst, N)
    nraw1, npost1 = _pna((s1, mx1, mn1), deg2, avg_d,
                         npost0, W_pna[1], bp[1], lnw[1], lnb[1])
    ns1 = _sc_gather(nraw1, src32)
    nd1 = _sc_gather(nraw1, dst32)
    ef2, eq2 = _edge_upd(False, ef1, eq1, ns1, nd1, None,
                         Wl1[1], Wl2[1], bl[1], Wq1[1], Wq2[1], Wq3[1],
                         bq[1], lnw[1], lnb[1])

    tef = jnp.concatenate([jnp.take(ef1, target_idx, axis=0),
                           jnp.take(ef2, target_idx, axis=0)], axis=1)
    teq = jnp.concatenate([jnp.take(eq1, target_idx, axis=0),
                           jnp.take(eq2, target_idx, axis=0)], axis=1)
    tn = jnp.take(src, target_idx)
    tnf = jnp.concatenate([jnp.take(npost0, tn, axis=0),
                           jnp.take(npost1, tn, axis=0)], axis=1)
    u, v, w = _head(tef, teq, tnf, W_ejk, b_ejk.reshape(1, D),
                    W_njk, b_njk.reshape(1, D), W_qjk, b_qjk.reshape(1, D),
                    W_fc)
    u = u[:, 0]
    v = v[:, 0]
    w = w[:, 0]
    right = u[0::2] + v[0::2] + w[1::2] + b_fc[0]
    left = u[1::2] + v[1::2] + w[0::2] + b_fc[0]
    return jnp.maximum(right, left)[:, None]


# speedup vs baseline: 4.9269x; 1.0116x over previous
import functools

import jax
import jax.numpy as jnp
from jax import lax
from jax.experimental import pallas as pl
from jax.experimental.pallas import tpu as pltpu
from jax.experimental.pallas import tpu_sc as plsc

D = 128
_INTERPRET = False


def _block(n, candidates):
    for c in candidates:
        if n % c == 0:
            return c
    return n


def _edge_block(E):
    return _block(E, (2000, 1600, 800, 400, 200, 80, 40, 16, 8))


def _node_block(N):
    return _block(N, (400, 200, 100, 50, 40, 16, 8))


def _ln(x, w, b):
    mu = jnp.mean(x, axis=-1, keepdims=True)
    var = jnp.mean((x - mu) ** 2, axis=-1, keepdims=True)
    return (x - mu) * jax.lax.rsqrt(var + 1e-5) * w + b


def _vspec(blk, imap):
    return pl.BlockSpec(blk, imap)


def _wspec(shape):
    return pl.BlockSpec(shape, lambda *a: (0,) * len(shape))


def _edge_msg_l0_body(tidx, re, eq, tq, Wx, bg, m):
    eb = re.shape[0]
    gid = jax.lax.broadcasted_iota(jnp.int32, (eb, tidx.shape[1]), 0)
    gid = gid + pl.program_id(0) * eb
    onehot = (gid == tidx[...]).astype(jnp.float32)
    ef0 = jnp.dot(onehot, tq[...], preferred_element_type=jnp.float32)
    x = ef0 * re[...] + eq[...]
    gx = jnp.dot(x, Wx[...], preferred_element_type=jnp.float32) + bg[...]
    z = jax.nn.sigmoid(gx[:, :D])
    n = jnp.tanh(gx[:, 2 * D:])
    m[...] = z * n


def _edge_msg_l1_body(re, eq, ef, h, Wx, Wh, bg, m):
    x = ef[...] * re[...] + eq[...]
    gx = jnp.dot(x, Wx[...], preferred_element_type=jnp.float32) + bg[...]
    gh = jnp.dot(h[...], Wh[...], preferred_element_type=jnp.float32)
    z = jax.nn.sigmoid(gx[:, :D] + gh[:, :D])
    r = jax.nn.sigmoid(gx[:, D:2 * D] + gh[:, D:2 * D])
    n = jnp.tanh(gx[:, 2 * D:] + r * gh[:, 2 * D:])
    m[...] = (1.0 - z) * h[...] + z * n


def _edge_msg_l0(re, eq, tq, tidx, Wx, bg):
    E = re.shape[0]
    eb = _edge_block(E)
    grid = E // eb
    t = tidx.shape[1]
    return pl.pallas_call(
        _edge_msg_l0_body,
        grid=(grid,),
        in_specs=[
            _wspec((1, t)),
            _vspec((eb, D), lambda i: (i, 0)),
            _vspec((eb, D), lambda i: (i, 0)),
            _wspec((t, D)),
            _wspec((D, 3 * D)),
            _wspec((1, 3 * D)),
        ],
        out_specs=_vspec((eb, D), lambda i: (i, 0)),
        out_shape=jax.ShapeDtypeStruct((E, D), jnp.float32),
        interpret=_INTERPRET,
    )(tidx, re, eq, tq, Wx, bg)


def _edge_msg_l1(re, eq, ef, h, Wx, Wh, bg):
    E = re.shape[0]
    eb = _edge_block(E)
    grid = E // eb
    return pl.pallas_call(
        _edge_msg_l1_body,
        grid=(grid,),
        in_specs=[
            _vspec((eb, D), lambda i: (i, 0)),
            _vspec((eb, D), lambda i: (i, 0)),
            _vspec((eb, D), lambda i: (i, 0)),
            _vspec((eb, D), lambda i: (i, 0)),
            _wspec((D, 3 * D)),
            _wspec((D, 3 * D)),
            _wspec((1, 3 * D)),
        ],
        out_specs=_vspec((eb, D), lambda i: (i, 0)),
        out_shape=jax.ShapeDtypeStruct((E, D), jnp.float32),
        interpret=_INTERPRET,
    )(re, eq, ef, h, Wx, Wh, bg)


def _pna_body(ssum, smax, smin, degr, avgr, prev, Wp, bp, lnw, lnb, raw,
              post):
    deg = degr[...]
    degc = jnp.maximum(deg, 1.0)
    pos = deg > 0
    mean = ssum[...] / degc
    mx = jnp.where(pos, smax[...], 0.0)
    mn = jnp.where(pos, smin[...], 0.0)
    agg = jnp.concatenate([mean, mx, mn], axis=1)
    ld = jnp.log(deg + 1.0)
    avg = avgr[...]
    amp = ld / avg
    att = avg / jnp.where(ld > 0, ld, 1.0)
    h_agg = jnp.concatenate([agg, agg * amp, agg * att], axis=1)
    r = jnp.dot(h_agg, Wp[...], preferred_element_type=jnp.float32) + bp[...]
    raw[...] = r
    post[...] = prev[...] + _ln(r, lnw[...], lnb[...])


def _pna(aggs, deg, avg_d, prev, Wp, bp, lnw, lnb):
    N = prev.shape[0]
    nb = _node_block(N)
    grid = N // nb
    return pl.pallas_call(
        _pna_body,
        grid=(grid,),
        in_specs=[
            _vspec((nb, D), lambda i: (i, 0)),
            _vspec((nb, D), lambda i: (i, 0)),
            _vspec((nb, D), lambda i: (i, 0)),
            _vspec((nb, 1), lambda i: (i, 0)),
            _wspec((1, 1)),
            _vspec((nb, D), lambda i: (i, 0)),
            _wspec((9 * D, D)),
            _wspec((1, D)),
            _wspec((1, D)),
            _wspec((1, D)),
        ],
        out_specs=[
            _vspec((nb, D), lambda i: (i, 0)),
            _vspec((nb, D), lambda i: (i, 0)),
        ],
        out_shape=[
            jax.ShapeDtypeStruct((N, D), jnp.float32),
            jax.ShapeDtypeStruct((N, D), jnp.float32),
        ],
        interpret=_INTERPRET,
    )(*aggs, deg, avg_d, prev, Wp, bp, lnw, lnb)


def _edge_upd_common(ef, eq, ns, nd, Wl1, Wl2, bl, Wq1, Wq2, Wq3, bq, lnw, lnb,
                     efo, eqo):
    g4 = (jnp.dot(ns, Wl1, preferred_element_type=jnp.float32)
          + jnp.dot(nd, Wl2, preferred_element_type=jnp.float32) + bl)
    gi = g4[:, :D]
    gf = g4[:, D:2 * D]
    go = g4[:, 2 * D:3 * D]
    gc = g4[:, 3 * D:]
    c = jax.nn.sigmoid(gf) * ef + jax.nn.sigmoid(gi) * jnp.tanh(gc)
    efn = jax.nn.sigmoid(go) * jnp.tanh(c)
    eqn = jnp.tanh(jnp.dot(eq, Wq1, preferred_element_type=jnp.float32)
                   + jnp.dot(ns, Wq2, preferred_element_type=jnp.float32)
                   + jnp.dot(nd, Wq3, preferred_element_type=jnp.float32) + bq)
    efo[...] = ef + _ln(efn, lnw, lnb)
    eqo[...] = eq + _ln(eqn, lnw, lnb)


def _edge_upd_l0_body(tidx, eq, ns, nd, tq, Wl1, Wl2, bl, Wq1, Wq2, Wq3, bq,
                      lnw, lnb, efo, eqo):
    eb = eq.shape[0]
    gid = jax.lax.broadcasted_iota(jnp.int32, (eb, tidx.shape[1]), 0)
    gid = gid + pl.program_id(0) * eb
    onehot = (gid == tidx[...]).astype(jnp.float32)
    ef = jnp.dot(onehot, tq[...], preferred_element_type=jnp.float32)
    _edge_upd_common(ef, eq[...], ns[...], nd[...], Wl1[...], Wl2[...],
                     bl[...], Wq1[...], Wq2[...], Wq3[...], bq[...],
                     lnw[...], lnb[...], efo, eqo)


def _edge_upd_l1_body(ef, eq, ns, nd, Wl1, Wl2, bl, Wq1, Wq2, Wq3, bq,
                      lnw, lnb, efo, eqo):
    _edge_upd_common(ef[...], eq[...], ns[...], nd[...], Wl1[...], Wl2[...],
                     bl[...], Wq1[...], Wq2[...], Wq3[...], bq[...],
                     lnw[...], lnb[...], efo, eqo)


def _edge_upd(layer0, ef_or_tidx, eq, ns, nd, tq, Wl1, Wl2, bl, Wq1, Wq2,
              Wq3, bq, lnw, lnb):
    E = eq.shape[0]
    eb = _edge_block(E)
    grid = E // eb
    espec = _vspec((eb, D), lambda i: (i, 0))
    wspecs = [
        _wspec((D, 4 * D)), _wspec((D, 4 * D)), _wspec((1, 4 * D)),
        _wspec((D, D)), _wspec((D, D)), _wspec((D, D)), _wspec((1, D)),
        _wspec((1, D)), _wspec((1, D)),
    ]
    outs = dict(
        out_specs=[espec, espec],
        out_shape=[jax.ShapeDtypeStruct((E, D), jnp.float32),
                   jax.ShapeDtypeStruct((E, D), jnp.float32)],
        interpret=_INTERPRET,
    )
    if layer0:
        t = ef_or_tidx.shape[1]
        return pl.pallas_call(
            _edge_upd_l0_body,
            grid=(grid,),
            in_specs=[_wspec((1, t)), espec, espec, espec, _wspec((t, D))]
            + wspecs,
            **outs,
        )(ef_or_tidx, eq, ns, nd, tq, Wl1, Wl2, bl, Wq1, Wq2, Wq3, bq,
          lnw, lnb)
    return pl.pallas_call(
        _edge_upd_l1_body,
        grid=(grid,),
        in_specs=[espec, espec, espec, espec] + wspecs,
        **outs,
    )(ef_or_tidx, eq, ns, nd, Wl1, Wl2, bl, Wq1, Wq2, Wq3, bq, lnw, lnb)


def _head_body(tef, teq, tnf, Wej, bej, Wnj, bnj, Wqj, bqj, Wfc, u, v, w):
    ef32 = jnp.dot(tef[...], Wej[...], preferred_element_type=jnp.float32) + bej[...]
    eq32 = jnp.dot(teq[...], Wqj[...], preferred_element_type=jnp.float32) + bqj[...]
    nf32 = jnp.dot(tnf[...], Wnj[...], preferred_element_type=jnp.float32) + bnj[...]
    A = Wfc[:D, :]
    Bp = Wfc[D:2 * D, :]
    C = Wfc[2 * D:3 * D, :]
    Dp = Wfc[3 * D:, :]
    u[...] = (jnp.dot(ef32, A, preferred_element_type=jnp.float32)
              + jnp.dot(eq32, Bp, preferred_element_type=jnp.float32))
    v[...] = jnp.dot(nf32, C, preferred_element_type=jnp.float32)
    w[...] = jnp.dot(nf32, Dp, preferred_element_type=jnp.float32)


def _head(tef, teq, tnf, Wej, bej, Wnj, bnj, Wqj, bqj, Wfc):
    T = tef.shape[0]
    K = tef.shape[1]
    return pl.pallas_call(
        _head_body,
        in_specs=[
            _wspec((T, K)), _wspec((T, K)), _wspec((T, K)),
            _wspec((K, D)), _wspec((1, D)),
            _wspec((K, D)), _wspec((1, D)),
            _wspec((K, D)), _wspec((1, D)),
            _wspec((4 * D, 1)),
        ],
        out_specs=[_wspec((T, 1)), _wspec((T, 1)), _wspec((T, 1))],
        out_shape=[jax.ShapeDtypeStruct((T, 1), jnp.float32)] * 3,
        interpret=_INTERPRET,
    )(tef, teq, tnf, Wej, bej, Wnj, bnj, Wqj, bqj, Wfc)


_GCH = 128


def _sc_mesh():
    return plsc.VectorSubcoreMesh(core_axis_name="c", subcore_axis_name="s")


def _sc_gather(table, idx):
    E2 = idx.shape[0]
    Dt = table.shape[1]
    assert E2 % _GCH == 0
    nch = E2 // _GCH
    info = plsc.get_sparse_core_info()
    NC, NS = info.num_cores, info.num_subcores
    NW = NC * NS
    bound = -(-nch // NW)

    @functools.partial(
        pl.kernel,
        mesh=_sc_mesh(),
        out_type=jax.ShapeDtypeStruct((E2, Dt), jnp.float32),
        scratch_types=[
            pltpu.VMEM((_GCH,), jnp.int32),
            pltpu.VMEM((_GCH, Dt), jnp.float32),
            pltpu.SemaphoreType.DMA,
        ],
    )
    def k(table_hbm, idx_hbm, out_hbm, idxb, rows, sem):
        wid = lax.axis_index("s") * NC + lax.axis_index("c")

        def body(j, carry):
            c = wid + j * NW

            @pl.when(c < nch)
            def _():
                base = c * _GCH
                pltpu.sync_copy(idx_hbm.at[pl.ds(base, _GCH)], idxb)
                pltpu.async_copy(table_hbm.at[idxb], rows, sem).wait()
                pltpu.sync_copy(rows, out_hbm.at[pl.ds(base, _GCH)])

            return carry

        lax.fori_loop(0, bound, body, 0)

    return k(table, idx)


def _segment_reduce(m, dst, N):
    ssum = jax.ops.segment_sum(m, dst, num_segments=N)
    smax = jax.ops.segment_max(m, dst, num_segments=N)
    smin = jax.ops.segment_min(m, dst, num_segments=N)
    return ssum, smax, smin


def kernel(edge_index, etype, target_idx, edge_gid, q_emb, W_qp, b_qp,
           rel_emb, Wx, Wh, b_g, W_pna, b_pna, W_lstm, b_lstm, W_eq, b_eq,
           ln_w, ln_b, W_ejk, b_ejk, W_njk, b_njk, W_qjk, b_qjk, W_fc, b_fc):
    E = etype.shape[0]
    N = 10000
    src = edge_index[0]
    dst = edge_index[1]
    T = target_idx.shape[0]
    B = T // 2

    queries = jnp.take(q_emb, jnp.take(etype, target_idx), axis=0)
    eqm = target_idx[None, :] == target_idx[:, None]
    later = jnp.triu(eqm, 1).any(axis=1)
    tidx_masked = jnp.where(later, -1, target_idx).astype(jnp.int32)[None, :]
    T2 = queries.reshape(B, 2 * D) @ W_qp + b_qp

    src32 = src.astype(jnp.int32)
    dst32 = dst.astype(jnp.int32)
    ones_e = jnp.ones((E,), jnp.float32)
    out_deg = jax.ops.segment_sum(ones_e, src, num_segments=N)
    deg = jax.ops.segment_sum(ones_e, dst, num_segments=N)
    avg_d = jnp.mean(jnp.log(out_deg + 1.0)).reshape(1, 1)
    deg2 = deg.reshape(N, 1)

    equery = _sc_gather(T2, edge_gid.astype(jnp.int32))
    etype32 = etype.astype(jnp.int32)
    re0 = _sc_gather(rel_emb[0], etype32)
    re1 = _sc_gather(rel_emb[1], etype32)

    Wl1 = [W_lstm[l, :D] for l in range(2)]
    Wl2 = [W_lstm[l, D:] for l in range(2)]
    Wq1 = [W_eq[l, :D] for l in range(2)]
    Wq2 = [W_eq[l, D:2 * D] for l in range(2)]
    Wq3 = [W_eq[l, 2 * D:] for l in range(2)]
    bg = b_g.reshape(2, 1, 3 * D)
    bl = b_lstm.reshape(2, 1, 4 * D)
    bq = b_eq.reshape(2, 1, D)
    bp = b_pna.reshape(2, 1, D)
    lnw = ln_w.reshape(2, 1, D)
    lnb = ln_b.reshape(2, 1, D)

    m0 = _edge_msg_l0(re0, equery, queries, tidx_masked, Wx[0], bg[0])
    s0, mx0, mn0 = _segment_reduce(m0, dst, N)
    nraw0, npost0 = _pna((s0, mx0, mn0), deg2, avg_d,
                         jnp.zeros((N, D), jnp.float32),
                         W_pna[0], bp[0], lnw[0], lnb[0])
    ns0 = _sc_gather(nraw0, src32)
    nd0 = _sc_gather(nraw0, dst32)
    ef1, eq1 = _edge_upd(True, tidx_masked, equery, ns0, nd0, queries,
                         Wl1[0], Wl2[0], bl[0], Wq1[0], Wq2[0], Wq3[0],
                         bq[0], lnw[0], lnb[0])

    h1 = _sc_gather(npost0, src32)
    m1 = _edge_msg_l1(re1, eq1, ef1, h1, Wx[1], Wh[1], bg[1])
    s1, mx1, mn1 = _segment_reduce(m1, dst, N)
    nraw1, npost1 = _pna((s1, mx1, mn1), deg2, avg_d,
                         npost0, W_pna[1], bp[1], lnw[1], lnb[1])
    ns1 = _sc_gather(nraw1, src32)
    nd1 = _sc_gather(nraw1, dst32)
    ef2, eq2 = _edge_upd(False, ef1, eq1, ns1, nd1, None,
                         Wl1[1], Wl2[1], bl[1], Wq1[1], Wq2[1], Wq3[1],
                         bq[1], lnw[1], lnb[1])

    tef = jnp.concatenate([jnp.take(ef1, target_idx, axis=0),
                           jnp.take(ef2, target_idx, axis=0)], axis=1)
    teq = jnp.concatenate([jnp.take(eq1, target_idx, axis=0),
                           jnp.take(eq2, target_idx, axis=0)], axis=1)
    tn = jnp.take(src, target_idx)
    tnf = jnp.concatenate([jnp.take(npost0, tn, axis=0),
                           jnp.take(npost1, tn, axis=0)], axis=1)
    u, v, w = _head(tef, teq, tnf, W_ejk, b_ejk.reshape(1, D),
                    W_njk, b_njk.reshape(1, D), W_qjk, b_qjk.reshape(1, D),
                    W_fc)
    u = u[:, 0]
    v = v[:, 0]
    w = w[:, 0]
    right = u[0::2] + v[0::2] + w[1::2] + b_fc[0]
    left = u[1::2] + v[1::2] + w[0::2] + b_fc[0]
    return jnp.maximum(right, left)[:, None]
